# baseline (speedup 1.0000x reference)
import functools

import jax
import jax.numpy as jnp
from jax import lax
from jax.experimental import pallas as pl
from jax.experimental.pallas import tpu as pltpu
from jax.experimental.pallas import tpu_sc as plsc

N_NODES = 10000
N_EDGES = 320000
IN_FEATS = 128
N_HIDDEN = 16
N_CLASSES = 16

NS = 16
CH = 128
GC = 512
SPG = GC // CH

NG = -(-N_EDGES // (NS * GC))
E_PT = NG * GC
EPAD = NS * E_PT
NCH = E_PT // CH

NPAD = -(-(N_NODES + 1) // (NS * 8)) * (NS * 8)
ROWS_PT = NPAD // NS


def _sc_gcn(h1pre, src3, dst3, binit, zeros):
  mesh = plsc.VectorSubcoreMesh(
      core_axis_name="c", subcore_axis_name="s", num_cores=1)

  @functools.partial(
      pl.kernel,
      mesh=mesh,
      compiler_params=pltpu.CompilerParams(use_tc_tiling_on_sc=False),
      out_type=(
          jax.ShapeDtypeStruct((NPAD, N_HIDDEN), jnp.float32),
          jax.ShapeDtypeStruct((NPAD, N_HIDDEN), jnp.float32),
      ),
      scratch_types=[
          pltpu.VMEM((NG, GC), jnp.int32),
          pltpu.VMEM((NCH, CH), jnp.int32),
          pltpu.VMEM((3, GC, N_HIDDEN), jnp.float32),
          pltpu.VMEM((ROWS_PT, N_HIDDEN), jnp.float32),
          pltpu.VMEM_SHARED((NPAD, N_HIDDEN), jnp.float32),
          pltpu.SemaphoreType.DMA,
          pltpu.SemaphoreType.DMA,
          pltpu.SemaphoreType.DMA,
      ],
  )
  def gcn(h1_hbm, src_hbm, dst_hbm, binit_hbm, zeros_hbm,
          agg2_hbm, h2_hbm,
          src_v, dst_v, rows_v, mid_v, accum_sh, gsem, ssem, zsem):
    s = lax.axis_index("s")
    acc_rows = pl.ds(s * ROWS_PT, ROWS_PT)

    def edge_pass(table_hbm):
      pltpu.async_copy(table_hbm.at[src_v.at[0]], rows_v.at[0], gsem)
      pltpu.async_copy(table_hbm.at[src_v.at[1]], rows_v.at[1], gsem)

      def body(g, carry):
        bsel = lax.rem(g, 3)
        prev = lax.rem(g + 2, 3)
        pltpu.make_async_copy(
            table_hbm.at[src_v.at[g]], rows_v.at[bsel], gsem).wait()
        @pl.when(g > 0)
        def _():
          for t in range(SPG):
            pltpu.make_async_copy(
                rows_v.at[prev, pl.ds(t * CH, CH)],
                accum_sh.at[dst_v.at[(g - 1) * SPG + t]], ssem).wait()
        @pl.when(g + 2 < NG)
        def _():
          pltpu.async_copy(
              table_hbm.at[src_v.at[g + 2]], rows_v.at[prev], gsem)
        for t in range(SPG):
          pltpu.async_copy(
              rows_v.at[bsel, pl.ds(t * CH, CH)],
              accum_sh.at[dst_v.at[g * SPG + t]], ssem, add=True)
        return carry

      lax.fori_loop(0, NG, body, 0)
      last = (NG - 1) % 3
      for t in range(SPG):
        pltpu.make_async_copy(
            rows_v.at[last, pl.ds(t * CH, CH)],
            accum_sh.at[dst_v.at[(NG - 1) * SPG + t]], ssem).wait()

    ic = pltpu.async_copy(binit_hbm.at[acc_rows], accum_sh.at[acc_rows], zsem)
    sc_ = pltpu.async_copy(src_hbm.at[s], src_v, gsem)
    dc = pltpu.async_copy(dst_hbm.at[s], dst_v, zsem)
    sc_.wait()
    dc.wait()
    ic.wait()
    plsc.subcore_barrier()

    edge_pass(h1_hbm)
    plsc.subcore_barrier()

    pltpu.sync_copy(accum_sh.at[acc_rows], mid_v)

    def relu_body(i, carry):
      mid_v[i] = jnp.maximum(mid_v[i], 0.0)
      return carry
    lax.fori_loop(0, ROWS_PT, relu_body, 0)
    zc = pltpu.async_copy(zeros_hbm.at[acc_rows], accum_sh.at[acc_rows], zsem)
    hc = pltpu.async_copy(mid_v, h2_hbm.at[acc_rows], gsem)
    zc.wait()
    hc.wait()
    plsc.subcore_barrier()

    edge_pass(h2_hbm)
    plsc.subcore_barrier()
    pltpu.sync_copy(accum_sh.at[acc_rows], agg2_hbm.at[acc_rows])

  return gcn(h1pre, src3, dst3, binit, zeros)


def _tc_project1(features, w1t):
  def body(x_ref, w_ref, o_ref):
    o_ref[...] = jnp.dot(x_ref[...], w_ref[...],
                         preferred_element_type=jnp.float32)
  return pl.pallas_call(
      body,
      out_shape=jax.ShapeDtypeStruct((N_NODES, N_HIDDEN), jnp.float32),
  )(features, w1t)


def _tc_final(agg2, w2t, b2):
  def body(p_ref, w_ref, b_ref, o_ref):
    o_ref[...] = jnp.dot(p_ref[:N_NODES], w_ref[...],
                         preferred_element_type=jnp.float32) + b_ref[...]
  return pl.pallas_call(
      body,
      out_shape=jax.ShapeDtypeStruct((N_NODES, N_CLASSES), jnp.float32),
  )(agg2, w2t, b2)


def kernel(features, edge_index, W1, b1, W2, b2):
  src = edge_index[0].astype(jnp.int32)
  dst = edge_index[1].astype(jnp.int32)
  src3 = jnp.concatenate(
      [src, jnp.zeros((EPAD - N_EDGES,), jnp.int32)]).reshape(NS, NG, GC)
  dst3 = jnp.concatenate(
      [dst, jnp.full((EPAD - N_EDGES,), N_NODES, jnp.int32)]
  ).reshape(NS, NCH, CH)
  binit = jnp.broadcast_to(b1, (NPAD, N_HIDDEN)).astype(jnp.float32)
  zeros = jnp.zeros((NPAD, N_HIDDEN), jnp.float32)

  h1pre = _tc_project1(features, W1.T)
  agg2, _ = _sc_gcn(h1pre, src3, dst3, binit, zeros)
  return _tc_final(agg2, W2.T, b2.reshape(1, N_CLASSES))


# speedup vs baseline: 1.5113x; 1.5113x over previous
import functools

import jax
import jax.numpy as jnp
from jax import lax
from jax.experimental import pallas as pl
from jax.experimental.pallas import tpu as pltpu
from jax.experimental.pallas import tpu_sc as plsc

N_NODES = 10000
N_EDGES = 320000
IN_FEATS = 128
N_HIDDEN = 16
N_CLASSES = 16

NC = 2
NS = 16
NW = NC * NS
CH = 128
GC = 512
SPG = GC // CH

NG = -(-N_EDGES // (NW * GC))
E_PT = NG * GC
EPAD = NW * E_PT
NCH = E_PT // CH

NPAD = -(-(N_NODES + 1) // (NS * 8)) * (NS * 8)
ROWS_PT = NPAD // NS


def _edge_loop(table_ref, src_v, dst_v, rows_v, accum_sh, gsem, ssem):
  pltpu.async_copy(table_ref.at[src_v.at[0]], rows_v.at[0], gsem)
  pltpu.async_copy(table_ref.at[src_v.at[1]], rows_v.at[1], gsem)

  def body(g, carry):
    bsel = lax.rem(g, 3)
    prev = lax.rem(g + 2, 3)
    pltpu.make_async_copy(
        table_ref.at[src_v.at[g]], rows_v.at[bsel], gsem).wait()
    @pl.when(g > 0)
    def _():
      for t in range(SPG):
        pltpu.make_async_copy(
            rows_v.at[prev, pl.ds(t * CH, CH)],
            accum_sh.at[dst_v.at[(g - 1) * SPG + t]], ssem).wait()
    @pl.when(g + 2 < NG)
    def _():
      pltpu.async_copy(
          table_ref.at[src_v.at[g + 2]], rows_v.at[prev], gsem)
    for t in range(SPG):
      pltpu.async_copy(
          rows_v.at[bsel, pl.ds(t * CH, CH)],
          accum_sh.at[dst_v.at[g * SPG + t]], ssem, add=True)
    return carry

  lax.fori_loop(0, NG, body, 0)
  last = (NG - 1) % 3
  for t in range(SPG):
    pltpu.make_async_copy(
        rows_v.at[last, pl.ds(t * CH, CH)],
        accum_sh.at[dst_v.at[(NG - 1) * SPG + t]], ssem).wait()


_SC_SCRATCH = [
    pltpu.VMEM((NG, GC), jnp.int32),
    pltpu.VMEM((NCH, CH), jnp.int32),
    pltpu.VMEM((3, GC, N_HIDDEN), jnp.float32),
    pltpu.VMEM_SHARED((NPAD, N_HIDDEN), jnp.float32),
    pltpu.SemaphoreType.DMA,
    pltpu.SemaphoreType.DMA,
    pltpu.SemaphoreType.DMA,
]

_MESH = dict(core_axis_name="c", subcore_axis_name="s")


def _sc_pass1(table, src3, dst3, init2):
  @functools.partial(
      pl.kernel,
      mesh=plsc.VectorSubcoreMesh(**_MESH),
      compiler_params=pltpu.CompilerParams(use_tc_tiling_on_sc=False),
      out_type=jax.ShapeDtypeStruct((NC, NPAD, N_HIDDEN), jnp.float32),
      scratch_types=_SC_SCRATCH,
  )
  def pass1(table_hbm, src_hbm, dst_hbm, init_hbm, out_hbm,
            src_v, dst_v, rows_v, accum_sh, gsem, ssem, zsem):
    c = lax.axis_index("c")
    s = lax.axis_index("s")
    wid = s * NC + c
    acc_rows = pl.ds(s * ROWS_PT, ROWS_PT)

    ic = pltpu.async_copy(init_hbm.at[c, acc_rows], accum_sh.at[acc_rows],
                          zsem)
    sc_ = pltpu.async_copy(src_hbm.at[wid], src_v, gsem)
    dc = pltpu.async_copy(dst_hbm.at[wid], dst_v, zsem)
    sc_.wait()
    dc.wait()
    ic.wait()
    plsc.subcore_barrier()

    _edge_loop(table_hbm, src_v, dst_v, rows_v, accum_sh, gsem, ssem)

    plsc.subcore_barrier()
    pltpu.sync_copy(accum_sh.at[acc_rows], out_hbm.at[c, acc_rows])

  return pass1(table, src3, dst3, init2)


def _sc_pass2(parts1, src3, dst3, zeros):
  @functools.partial(
      pl.kernel,
      mesh=plsc.VectorSubcoreMesh(**_MESH),
      compiler_params=pltpu.CompilerParams(use_tc_tiling_on_sc=False),
      out_type=jax.ShapeDtypeStruct((NC, NPAD, N_HIDDEN), jnp.float32),
      scratch_types=_SC_SCRATCH + [
          pltpu.VMEM((ROWS_PT, N_HIDDEN), jnp.float32),
          pltpu.VMEM((ROWS_PT, N_HIDDEN), jnp.float32),
          pltpu.VMEM_SHARED((NPAD, N_HIDDEN), jnp.float32),
      ],
  )
  def pass2(parts_hbm, src_hbm, dst_hbm, zeros_hbm, out_hbm,
            src_v, dst_v, rows_v, accum_sh, gsem, ssem, zsem,
            h2_v, p1_v, h2_sh):
    c = lax.axis_index("c")
    s = lax.axis_index("s")
    wid = s * NC + c
    acc_rows = pl.ds(s * ROWS_PT, ROWS_PT)

    zc = pltpu.async_copy(zeros_hbm.at[acc_rows], accum_sh.at[acc_rows],
                          zsem)
    sc_ = pltpu.async_copy(src_hbm.at[wid], src_v, gsem)
    dc = pltpu.async_copy(dst_hbm.at[wid], dst_v, zsem)
    p0c = pltpu.async_copy(parts_hbm.at[0, acc_rows], h2_v, gsem)
    p1c = pltpu.async_copy(parts_hbm.at[1, acc_rows], p1_v, ssem)
    p0c.wait()
    p1c.wait()

    def relu_body(i, carry):
      h2_v[i] = jnp.maximum(h2_v[i] + p1_v[i], 0.0)
      return carry
    lax.fori_loop(0, ROWS_PT, relu_body, 0)
    pltpu.sync_copy(h2_v, h2_sh.at[acc_rows])

    sc_.wait()
    dc.wait()
    zc.wait()
    plsc.subcore_barrier()

    _edge_loop(h2_sh, src_v, dst_v, rows_v, accum_sh, gsem, ssem)

    plsc.subcore_barrier()
    pltpu.sync_copy(accum_sh.at[acc_rows], out_hbm.at[c, acc_rows])

  return pass2(parts1, src3, dst3, zeros)


def _tc_project1(features, w1t):
  def body(x_ref, w_ref, o_ref):
    o_ref[...] = jnp.dot(x_ref[...], w_ref[...],
                         preferred_element_type=jnp.float32)
  return pl.pallas_call(
      body,
      out_shape=jax.ShapeDtypeStruct((N_NODES, N_HIDDEN), jnp.float32),
  )(features, w1t)


def _tc_final(parts2, w2t, b2):
  def body(p_ref, w_ref, b_ref, o_ref):
    agg2 = p_ref[0, :N_NODES] + p_ref[1, :N_NODES]
    o_ref[...] = jnp.dot(agg2, w_ref[...],
                         preferred_element_type=jnp.float32) + b_ref[...]
  return pl.pallas_call(
      body,
      out_shape=jax.ShapeDtypeStruct((N_NODES, N_CLASSES), jnp.float32),
  )(parts2, w2t, b2)


def kernel(features, edge_index, W1, b1, W2, b2):
  src = edge_index[0].astype(jnp.int32)
  dst = edge_index[1].astype(jnp.int32)
  src3 = jnp.concatenate(
      [src, jnp.zeros((EPAD - N_EDGES,), jnp.int32)]).reshape(NW, NG, GC)
  dst3 = jnp.concatenate(
      [dst, jnp.full((EPAD - N_EDGES,), N_NODES, jnp.int32)]
  ).reshape(NW, NCH, CH)
  init2 = jnp.stack([
      jnp.broadcast_to(b1, (NPAD, N_HIDDEN)).astype(jnp.float32),
      jnp.zeros((NPAD, N_HIDDEN), jnp.float32),
  ])
  zeros = jnp.zeros((NPAD, N_HIDDEN), jnp.float32)

  h1pre = _tc_project1(features, W1.T)
  parts1 = _sc_pass1(h1pre, src3, dst3, init2)
  parts2 = _sc_pass2(parts1, src3, dst3, zeros)
  return _tc_final(parts2, W2.T, b2.reshape(1, N_CLASSES))


# speedup vs baseline: 1.9518x; 1.2915x over previous
import functools

import jax
import jax.numpy as jnp
from jax import lax
from jax.experimental import pallas as pl
from jax.experimental.pallas import tpu as pltpu
from jax.experimental.pallas import tpu_sc as plsc

N_NODES = 10000
N_EDGES = 320000
IN_FEATS = 128
N_HIDDEN = 16
N_CLASSES = 16

NC = 2
NS = 16
NW = NC * NS
CH = 128
GC = 512
SPG = GC // CH

NG = -(-N_EDGES // (NW * GC))
E_PT = NG * GC
EPAD = NW * E_PT
NCH = E_PT // CH

NPAD = -(-(N_NODES + 1) // (NS * 8)) * (NS * 8)
ROWS_PT = NPAD // NS


def _edge_loop(table_ref, src_v, dst_v, rows_v, accum_sh, gsem, ssem):
  pltpu.async_copy(table_ref.at[src_v.at[0]], rows_v.at[0], gsem)
  pltpu.async_copy(table_ref.at[src_v.at[1]], rows_v.at[1], gsem)

  def body(g, carry):
    bsel = lax.rem(g, 3)
    prev = lax.rem(g + 2, 3)
    pltpu.make_async_copy(
        table_ref.at[src_v.at[g]], rows_v.at[bsel], gsem).wait()
    @pl.when(g > 0)
    def _():
      for t in range(SPG):
        pltpu.make_async_copy(
            rows_v.at[prev, pl.ds(t * CH, CH)],
            accum_sh.at[dst_v.at[(g - 1) * SPG + t]], ssem).wait()
    @pl.when(g + 2 < NG)
    def _():
      pltpu.async_copy(
          table_ref.at[src_v.at[g + 2]], rows_v.at[prev], gsem)
    for t in range(SPG):
      pltpu.async_copy(
          rows_v.at[bsel, pl.ds(t * CH, CH)],
          accum_sh.at[dst_v.at[g * SPG + t]], ssem, add=True)
    return carry

  lax.fori_loop(0, NG, body, 0)
  last = (NG - 1) % 3
  for t in range(SPG):
    pltpu.make_async_copy(
        rows_v.at[last, pl.ds(t * CH, CH)],
        accum_sh.at[dst_v.at[(NG - 1) * SPG + t]], ssem).wait()


_SC_SCRATCH = [
    pltpu.VMEM((NG, GC), jnp.int32),
    pltpu.VMEM((NCH, CH), jnp.int32),
    pltpu.VMEM((3, GC, N_HIDDEN), jnp.float32),
    pltpu.VMEM_SHARED((NPAD, N_HIDDEN), jnp.float32),
    pltpu.SemaphoreType.DMA,
    pltpu.SemaphoreType.DMA,
    pltpu.SemaphoreType.DMA,
]

_MESH = dict(core_axis_name="c", subcore_axis_name="s")


def _sc_pass1(table, src3, dst3, init2):
  @functools.partial(
      pl.kernel,
      mesh=plsc.VectorSubcoreMesh(**_MESH),
      compiler_params=pltpu.CompilerParams(use_tc_tiling_on_sc=False),
      out_type=jax.ShapeDtypeStruct((NC, NPAD, N_HIDDEN), jnp.float32),
      scratch_types=_SC_SCRATCH + [
          pltpu.VMEM_SHARED((NPAD, N_HIDDEN), jnp.float32),
      ],
  )
  def pass1(table_hbm, src_hbm, dst_hbm, init_hbm, out_hbm,
            src_v, dst_v, rows_v, accum_sh, gsem, ssem, zsem, tab_sh):
    c = lax.axis_index("c")
    s = lax.axis_index("s")
    wid = s * NC + c
    acc_rows = pl.ds(s * ROWS_PT, ROWS_PT)

    ic = pltpu.async_copy(init_hbm.at[c, acc_rows], accum_sh.at[acc_rows],
                          zsem)
    sc_ = pltpu.async_copy(src_hbm.at[wid], src_v, gsem)
    dc = pltpu.async_copy(dst_hbm.at[wid], dst_v, zsem)
    tc_ = pltpu.async_copy(table_hbm.at[acc_rows], tab_sh.at[acc_rows], ssem)
    sc_.wait()
    dc.wait()
    ic.wait()
    tc_.wait()
    plsc.subcore_barrier()

    _edge_loop(tab_sh, src_v, dst_v, rows_v, accum_sh, gsem, ssem)

    plsc.subcore_barrier()
    pltpu.sync_copy(accum_sh.at[acc_rows], out_hbm.at[c, acc_rows])

  return pass1(table, src3, dst3, init2)


def _sc_pass2(parts1, src3, dst3, zeros):
  @functools.partial(
      pl.kernel,
      mesh=plsc.VectorSubcoreMesh(**_MESH),
      compiler_params=pltpu.CompilerParams(use_tc_tiling_on_sc=False),
      out_type=jax.ShapeDtypeStruct((NC, NPAD, N_HIDDEN), jnp.float32),
      scratch_types=_SC_SCRATCH + [
          pltpu.VMEM((ROWS_PT, N_HIDDEN), jnp.float32),
          pltpu.VMEM((ROWS_PT, N_HIDDEN), jnp.float32),
          pltpu.VMEM_SHARED((NPAD, N_HIDDEN), jnp.float32),
      ],
  )
  def pass2(parts_hbm, src_hbm, dst_hbm, zeros_hbm, out_hbm,
            src_v, dst_v, rows_v, accum_sh, gsem, ssem, zsem,
            h2_v, p1_v, h2_sh):
    c = lax.axis_index("c")
    s = lax.axis_index("s")
    wid = s * NC + c
    acc_rows = pl.ds(s * ROWS_PT, ROWS_PT)

    zc = pltpu.async_copy(zeros_hbm.at[acc_rows], accum_sh.at[acc_rows],
                          zsem)
    sc_ = pltpu.async_copy(src_hbm.at[wid], src_v, gsem)
    dc = pltpu.async_copy(dst_hbm.at[wid], dst_v, zsem)
    p0c = pltpu.async_copy(parts_hbm.at[0, acc_rows], h2_v, gsem)
    p1c = pltpu.async_copy(parts_hbm.at[1, acc_rows], p1_v, ssem)
    p0c.wait()
    p1c.wait()

    def relu_body(i, carry):
      h2_v[i] = jnp.maximum(h2_v[i] + p1_v[i], 0.0)
      return carry
    lax.fori_loop(0, ROWS_PT, relu_body, 0)
    pltpu.sync_copy(h2_v, h2_sh.at[acc_rows])

    sc_.wait()
    dc.wait()
    zc.wait()
    plsc.subcore_barrier()

    _edge_loop(h2_sh, src_v, dst_v, rows_v, accum_sh, gsem, ssem)

    plsc.subcore_barrier()
    pltpu.sync_copy(accum_sh.at[acc_rows], out_hbm.at[c, acc_rows])

  return pass2(parts1, src3, dst3, zeros)


def _tc_project1(features, w1t):
  def body(x_ref, w_ref, o_ref):
    o_ref[...] = jnp.zeros_like(o_ref)
    o_ref[:N_NODES] = jnp.dot(x_ref[...], w_ref[...],
                              preferred_element_type=jnp.float32)
  return pl.pallas_call(
      body,
      out_shape=jax.ShapeDtypeStruct((NPAD, N_HIDDEN), jnp.float32),
  )(features, w1t)


def _tc_final(parts2, w2t, b2):
  def body(p_ref, w_ref, b_ref, o_ref):
    agg2 = p_ref[0, :N_NODES] + p_ref[1, :N_NODES]
    o_ref[...] = jnp.dot(agg2, w_ref[...],
                         preferred_element_type=jnp.float32) + b_ref[...]
  return pl.pallas_call(
      body,
      out_shape=jax.ShapeDtypeStruct((N_NODES, N_CLASSES), jnp.float32),
  )(parts2, w2t, b2)


def kernel(features, edge_index, W1, b1, W2, b2):
  src = edge_index[0].astype(jnp.int32)
  dst = edge_index[1].astype(jnp.int32)
  src3 = jnp.concatenate(
      [src, jnp.zeros((EPAD - N_EDGES,), jnp.int32)]).reshape(NW, NG, GC)
  dst3 = jnp.concatenate(
      [dst, jnp.full((EPAD - N_EDGES,), N_NODES, jnp.int32)]
  ).reshape(NW, NCH, CH)
  init2 = jnp.stack([
      jnp.broadcast_to(b1, (NPAD, N_HIDDEN)).astype(jnp.float32),
      jnp.zeros((NPAD, N_HIDDEN), jnp.float32),
  ])
  zeros = jnp.zeros((NPAD, N_HIDDEN), jnp.float32)

  h1pre = _tc_project1(features, W1.T)
  parts1 = _sc_pass1(h1pre, src3, dst3, init2)
  parts2 = _sc_pass2(parts1, src3, dst3, zeros)
  return _tc_final(parts2, W2.T, b2.reshape(1, N_CLASSES))


# speedup vs baseline: 1.9910x; 1.0201x over previous
import functools

import jax
import jax.numpy as jnp
from jax import lax
from jax.experimental import pallas as pl
from jax.experimental.pallas import tpu as pltpu
from jax.experimental.pallas import tpu_sc as plsc

N_NODES = 10000
N_EDGES = 320000
IN_FEATS = 128
N_HIDDEN = 16
N_CLASSES = 16

NC = 2
NS = 16
NW = NC * NS
CH = 128
GC = 512
SPG = GC // CH

NG = -(-N_EDGES // (NW * GC))
E_PT = NG * GC
EPAD = NW * E_PT
NCH = E_PT // CH

NPAD = -(-(N_NODES + 1) // (NS * 8)) * (NS * 8)
ROWS_PT = NPAD // NS


def _edge_loop(table_ref, src_v, dst_v, rows_v, accum_sh, gsem, ssem):
  pltpu.async_copy(table_ref.at[src_v.at[0]], rows_v.at[0], gsem)
  pltpu.async_copy(table_ref.at[src_v.at[1]], rows_v.at[1], gsem)

  def body(g, carry):
    bsel = lax.rem(g, 4)
    old = lax.rem(g + 2, 4)
    pltpu.make_async_copy(
        table_ref.at[src_v.at[g]], rows_v.at[bsel], gsem).wait()
    @pl.when(g > 1)
    def _():
      for t in range(SPG):
        pltpu.make_async_copy(
            rows_v.at[old, pl.ds(t * CH, CH)],
            accum_sh.at[dst_v.at[(g - 2) * SPG + t]], ssem).wait()
    @pl.when(g + 2 < NG)
    def _():
      pltpu.async_copy(
          table_ref.at[src_v.at[g + 2]], rows_v.at[old], gsem)
    for t in range(SPG):
      pltpu.async_copy(
          rows_v.at[bsel, pl.ds(t * CH, CH)],
          accum_sh.at[dst_v.at[g * SPG + t]], ssem, add=True)
    return carry

  lax.fori_loop(0, NG, body, 0)
  for g in (NG - 2, NG - 1):
    for t in range(SPG):
      pltpu.make_async_copy(
          rows_v.at[g % 4, pl.ds(t * CH, CH)],
          accum_sh.at[dst_v.at[g * SPG + t]], ssem).wait()


_SC_SCRATCH = [
    pltpu.VMEM((NG, GC), jnp.int32),
    pltpu.VMEM((NCH, CH), jnp.int32),
    pltpu.VMEM((4, GC, N_HIDDEN), jnp.float32),
    pltpu.VMEM_SHARED((NPAD, N_HIDDEN), jnp.float32),
    pltpu.SemaphoreType.DMA,
    pltpu.SemaphoreType.DMA,
    pltpu.SemaphoreType.DMA,
]

_MESH = dict(core_axis_name="c", subcore_axis_name="s")


def _sc_pass1(table, src3, dst3, init2):
  @functools.partial(
      pl.kernel,
      mesh=plsc.VectorSubcoreMesh(**_MESH),
      compiler_params=pltpu.CompilerParams(use_tc_tiling_on_sc=False),
      out_type=jax.ShapeDtypeStruct((NC, NPAD, N_HIDDEN), jnp.float32),
      scratch_types=_SC_SCRATCH + [
          pltpu.VMEM_SHARED((NPAD, N_HIDDEN), jnp.float32),
      ],
  )
  def pass1(table_hbm, src_hbm, dst_hbm, init_hbm, out_hbm,
            src_v, dst_v, rows_v, accum_sh, gsem, ssem, zsem, tab_sh):
    c = lax.axis_index("c")
    s = lax.axis_index("s")
    wid = s * NC + c
    acc_rows = pl.ds(s * ROWS_PT, ROWS_PT)

    ic = pltpu.async_copy(init_hbm.at[c, acc_rows], accum_sh.at[acc_rows],
                          zsem)
    sc_ = pltpu.async_copy(src_hbm.at[wid], src_v, gsem)
    dc = pltpu.async_copy(dst_hbm.at[wid], dst_v, zsem)
    tc_ = pltpu.async_copy(table_hbm.at[acc_rows], tab_sh.at[acc_rows], ssem)
    sc_.wait()
    dc.wait()
    ic.wait()
    tc_.wait()
    plsc.subcore_barrier()

    _edge_loop(tab_sh, src_v, dst_v, rows_v, accum_sh, gsem, ssem)

    plsc.subcore_barrier()
    pltpu.sync_copy(accum_sh.at[acc_rows], out_hbm.at[c, acc_rows])

  return pass1(table, src3, dst3, init2)


def _sc_pass2(parts1, src3, dst3, zeros):
  @functools.partial(
      pl.kernel,
      mesh=plsc.VectorSubcoreMesh(**_MESH),
      compiler_params=pltpu.CompilerParams(use_tc_tiling_on_sc=False),
      out_type=jax.ShapeDtypeStruct((NC, NPAD, N_HIDDEN), jnp.float32),
      scratch_types=_SC_SCRATCH + [
          pltpu.VMEM((ROWS_PT, N_HIDDEN), jnp.float32),
          pltpu.VMEM((ROWS_PT, N_HIDDEN), jnp.float32),
          pltpu.VMEM_SHARED((NPAD, N_HIDDEN), jnp.float32),
      ],
  )
  def pass2(parts_hbm, src_hbm, dst_hbm, zeros_hbm, out_hbm,
            src_v, dst_v, rows_v, accum_sh, gsem, ssem, zsem,
            h2_v, p1_v, h2_sh):
    c = lax.axis_index("c")
    s = lax.axis_index("s")
    wid = s * NC + c
    acc_rows = pl.ds(s * ROWS_PT, ROWS_PT)

    zc = pltpu.async_copy(zeros_hbm.at[acc_rows], accum_sh.at[acc_rows],
                          zsem)
    sc_ = pltpu.async_copy(src_hbm.at[wid], src_v, gsem)
    dc = pltpu.async_copy(dst_hbm.at[wid], dst_v, zsem)
    p0c = pltpu.async_copy(parts_hbm.at[0, acc_rows], h2_v, gsem)
    p1c = pltpu.async_copy(parts_hbm.at[1, acc_rows], p1_v, ssem)
    p0c.wait()
    p1c.wait()

    def relu_body(i, carry):
      for u in range(8):
        r = i * 8 + u
        h2_v[r] = jnp.maximum(h2_v[r] + p1_v[r], 0.0)
      return carry
    lax.fori_loop(0, ROWS_PT // 8, relu_body, 0)
    pltpu.sync_copy(h2_v, h2_sh.at[acc_rows])

    sc_.wait()
    dc.wait()
    zc.wait()
    plsc.subcore_barrier()

    _edge_loop(h2_sh, src_v, dst_v, rows_v, accum_sh, gsem, ssem)

    plsc.subcore_barrier()
    pltpu.sync_copy(accum_sh.at[acc_rows], out_hbm.at[c, acc_rows])

  return pass2(parts1, src3, dst3, zeros)


def _tc_project1(features, w1t):
  def body(x_ref, w_ref, o_ref):
    o_ref[...] = jnp.zeros_like(o_ref)
    o_ref[:N_NODES] = jnp.dot(x_ref[...], w_ref[...],
                              preferred_element_type=jnp.float32)
  return pl.pallas_call(
      body,
      out_shape=jax.ShapeDtypeStruct((NPAD, N_HIDDEN), jnp.float32),
  )(features, w1t)


def _tc_final(parts2, w2t, b2):
  def body(p_ref, w_ref, b_ref, o_ref):
    agg2 = p_ref[0, :N_NODES] + p_ref[1, :N_NODES]
    o_ref[...] = jnp.dot(agg2, w_ref[...],
                         preferred_element_type=jnp.float32) + b_ref[...]
  return pl.pallas_call(
      body,
      out_shape=jax.ShapeDtypeStruct((N_NODES, N_CLASSES), jnp.float32),
  )(parts2, w2t, b2)


def kernel(features, edge_index, W1, b1, W2, b2):
  src = edge_index[0].astype(jnp.int32)
  dst = edge_index[1].astype(jnp.int32)
  src3 = jnp.concatenate(
      [src, jnp.zeros((EPAD - N_EDGES,), jnp.int32)]).reshape(NW, NG, GC)
  dst3 = jnp.concatenate(
      [dst, jnp.full((EPAD - N_EDGES,), N_NODES, jnp.int32)]
  ).reshape(NW, NCH, CH)
  init2 = jnp.stack([
      jnp.broadcast_to(b1, (NPAD, N_HIDDEN)).astype(jnp.float32),
      jnp.zeros((NPAD, N_HIDDEN), jnp.float32),
  ])
  zeros = jnp.zeros((NPAD, N_HIDDEN), jnp.float32)

  h1pre = _tc_project1(features, W1.T)
  parts1 = _sc_pass1(h1pre, src3, dst3, init2)
  parts2 = _sc_pass2(parts1, src3, dst3, zeros)
  return _tc_final(parts2, W2.T, b2.reshape(1, N_CLASSES))


# speedup vs baseline: 1.9985x; 1.0038x over previous
import functools

import jax
import jax.numpy as jnp
from jax import lax
from jax.experimental import pallas as pl
from jax.experimental.pallas import tpu as pltpu
from jax.experimental.pallas import tpu_sc as plsc

N_NODES = 10000
N_EDGES = 320000
IN_FEATS = 128
N_HIDDEN = 16
N_CLASSES = 16

NC = 2
NS = 16
NW = NC * NS
CH = 128
GC = 512
SPG = GC // CH

NG = -(-N_EDGES // (NW * GC))
E_PT = NG * GC
EPAD = NW * E_PT
NCH = E_PT // CH

NPAD = -(-(N_NODES + 1) // (NS * 8)) * (NS * 8)
ROWS_PT = NPAD // NS


def _edge_loop(table_ref, src_v, dst_v, rows_v, accum_sh, gsem, ssem):
  pltpu.async_copy(table_ref.at[src_v.at[0]], rows_v.at[0], gsem)
  pltpu.async_copy(table_ref.at[src_v.at[1]], rows_v.at[1], gsem)

  def body(g, carry):
    bsel = lax.rem(g, 3)
    prev = lax.rem(g + 2, 3)
    pltpu.make_async_copy(
        table_ref.at[src_v.at[g]], rows_v.at[bsel], gsem).wait()
    @pl.when(g > 0)
    def _():
      for t in range(SPG):
        pltpu.make_async_copy(
            rows_v.at[prev, pl.ds(t * CH, CH)],
            accum_sh.at[dst_v.at[(g - 1) * SPG + t]], ssem).wait()
    @pl.when(g + 2 < NG)
    def _():
      pltpu.async_copy(
          table_ref.at[src_v.at[g + 2]], rows_v.at[prev], gsem)
    for t in range(SPG):
      pltpu.async_copy(
          rows_v.at[bsel, pl.ds(t * CH, CH)],
          accum_sh.at[dst_v.at[g * SPG + t]], ssem, add=True)
    return carry

  lax.fori_loop(0, NG, body, 0)
  last = (NG - 1) % 3
  for t in range(SPG):
    pltpu.make_async_copy(
        rows_v.at[last, pl.ds(t * CH, CH)],
        accum_sh.at[dst_v.at[(NG - 1) * SPG + t]], ssem).wait()


_SC_SCRATCH = [
    pltpu.VMEM((NG, GC), jnp.int32),
    pltpu.VMEM((NCH, CH), jnp.int32),
    pltpu.VMEM((3, GC, N_HIDDEN), jnp.float32),
    pltpu.VMEM_SHARED((NPAD, N_HIDDEN), jnp.float32),
    pltpu.SemaphoreType.DMA,
    pltpu.SemaphoreType.DMA,
    pltpu.SemaphoreType.DMA,
]

_MESH = dict(core_axis_name="c", subcore_axis_name="s")


def _sc_pass1(table, src3, dst3, init2):
  @functools.partial(
      pl.kernel,
      mesh=plsc.VectorSubcoreMesh(**_MESH),
      compiler_params=pltpu.CompilerParams(use_tc_tiling_on_sc=False),
      out_type=jax.ShapeDtypeStruct((NC, NPAD, N_HIDDEN), jnp.float32),
      scratch_types=_SC_SCRATCH + [
          pltpu.VMEM_SHARED((NPAD, N_HIDDEN), jnp.float32),
      ],
  )
  def pass1(table_hbm, src_hbm, dst_hbm, init_hbm, out_hbm,
            src_v, dst_v, rows_v, accum_sh, gsem, ssem, zsem, tab_sh):
    c = lax.axis_index("c")
    s = lax.axis_index("s")
    wid = s * NC + c
    acc_rows = pl.ds(s * ROWS_PT, ROWS_PT)

    ic = pltpu.async_copy(init_hbm.at[c, acc_rows], accum_sh.at[acc_rows],
                          zsem)
    sc_ = pltpu.async_copy(src_hbm.at[wid], src_v, gsem)
    dc = pltpu.async_copy(dst_hbm.at[wid], dst_v, zsem)
    tc_ = pltpu.async_copy(table_hbm.at[acc_rows], tab_sh.at[acc_rows], ssem)
    sc_.wait()
    dc.wait()
    ic.wait()
    tc_.wait()
    plsc.subcore_barrier()

    _edge_loop(tab_sh, src_v, dst_v, rows_v, accum_sh, gsem, ssem)

    plsc.subcore_barrier()
    pltpu.sync_copy(accum_sh.at[acc_rows], out_hbm.at[c, acc_rows])

  return pass1(table, src3, dst3, init2)


def _sc_pass2(parts1, src3, dst3, zeros):
  @functools.partial(
      pl.kernel,
      mesh=plsc.VectorSubcoreMesh(**_MESH),
      compiler_params=pltpu.CompilerParams(use_tc_tiling_on_sc=False),
      out_type=jax.ShapeDtypeStruct((NC, NPAD, N_HIDDEN), jnp.float32),
      scratch_types=_SC_SCRATCH + [
          pltpu.VMEM((ROWS_PT, N_HIDDEN), jnp.float32),
          pltpu.VMEM((ROWS_PT, N_HIDDEN), jnp.float32),
          pltpu.VMEM_SHARED((NPAD, N_HIDDEN), jnp.float32),
      ],
  )
  def pass2(parts_hbm, src_hbm, dst_hbm, zeros_hbm, out_hbm,
            src_v, dst_v, rows_v, accum_sh, gsem, ssem, zsem,
            h2_v, p1_v, h2_sh):
    c = lax.axis_index("c")
    s = lax.axis_index("s")
    wid = s * NC + c
    acc_rows = pl.ds(s * ROWS_PT, ROWS_PT)

    zc = pltpu.async_copy(zeros_hbm.at[acc_rows], accum_sh.at[acc_rows],
                          zsem)
    sc_ = pltpu.async_copy(src_hbm.at[wid], src_v, gsem)
    dc = pltpu.async_copy(dst_hbm.at[wid], dst_v, zsem)
    p0c = pltpu.async_copy(parts_hbm.at[0, acc_rows], h2_v, gsem)
    p1c = pltpu.async_copy(parts_hbm.at[1, acc_rows], p1_v, ssem)
    p0c.wait()
    p1c.wait()

    def relu_body(i, carry):
      for u in range(8):
        r = i * 8 + u
        h2_v[r] = jnp.maximum(h2_v[r] + p1_v[r], 0.0)
      return carry
    lax.fori_loop(0, ROWS_PT // 8, relu_body, 0)
    pltpu.sync_copy(h2_v, h2_sh.at[acc_rows])

    sc_.wait()
    dc.wait()
    zc.wait()
    plsc.subcore_barrier()

    _edge_loop(h2_sh, src_v, dst_v, rows_v, accum_sh, gsem, ssem)

    plsc.subcore_barrier()
    pltpu.sync_copy(accum_sh.at[acc_rows], out_hbm.at[c, acc_rows])

  return pass2(parts1, src3, dst3, zeros)


def _tc_project1(features, w1t):
  def body(x_ref, w_ref, o_ref):
    o_ref[...] = jnp.zeros_like(o_ref)
    o_ref[:N_NODES] = jnp.dot(x_ref[...], w_ref[...],
                              preferred_element_type=jnp.float32)
  return pl.pallas_call(
      body,
      out_shape=jax.ShapeDtypeStruct((NPAD, N_HIDDEN), jnp.float32),
  )(features, w1t)


def _tc_final(parts2, w2t, b2):
  def body(p_ref, w_ref, b_ref, o_ref):
    agg2 = p_ref[0, :N_NODES] + p_ref[1, :N_NODES]
    o_ref[...] = jnp.dot(agg2, w_ref[...],
                         preferred_element_type=jnp.float32) + b_ref[...]
  return pl.pallas_call(
      body,
      out_shape=jax.ShapeDtypeStruct((N_NODES, N_CLASSES), jnp.float32),
  )(parts2, w2t, b2)


def kernel(features, edge_index, W1, b1, W2, b2):
  src = edge_index[0].astype(jnp.int32)
  dst = edge_index[1].astype(jnp.int32)
  src3 = jnp.concatenate(
      [src, jnp.zeros((EPAD - N_EDGES,), jnp.int32)]).reshape(NW, NG, GC)
  dst3 = jnp.concatenate(
      [dst, jnp.full((EPAD - N_EDGES,), N_NODES, jnp.int32)]
  ).reshape(NW, NCH, CH)
  init2 = jnp.stack([
      jnp.broadcast_to(b1, (NPAD, N_HIDDEN)).astype(jnp.float32),
      jnp.zeros((NPAD, N_HIDDEN), jnp.float32),
  ])
  zeros = jnp.zeros((NPAD, N_HIDDEN), jnp.float32)

  h1pre = _tc_project1(features, W1.T)
  parts1 = _sc_pass1(h1pre, src3, dst3, init2)
  parts2 = _sc_pass2(parts1, src3, dst3, zeros)
  return _tc_final(parts2, W2.T, b2.reshape(1, N_CLASSES))
